# Initial kernel scaffold; baseline (speedup 1.0000x reference)
import jax
import jax.numpy as jnp
from jax.experimental import pallas as pl


def kernel(q, k, weight):
    raise NotImplementedError("write your pallas kernel here")


# speedup vs baseline: 1.6178x; 1.6178x over previous
import functools

import jax
import jax.numpy as jnp
from jax import lax
from jax.experimental import pallas as pl
from jax.experimental.pallas import tpu as pltpu
from jax.experimental.pallas import tpu_sc as plsc

_NUM_CORES = 2
_NUM_SUBCORES = 16


def kernel(q, k, weight):
    q_len = q.shape[0]
    k_len = k.shape[0]
    max_len, d = weight.shape
    offset = max_len // 2 + max_len % 2

    n_workers = _NUM_CORES * _NUM_SUBCORES
    per_worker = q_len // n_workers
    slice_words = k_len * d

    mesh = plsc.VectorSubcoreMesh(core_axis_name="c", subcore_axis_name="s")

    @functools.partial(
        pl.kernel,
        out_type=jax.ShapeDtypeStruct((q_len * k_len * d,), jnp.float32),
        mesh=mesh,
        scratch_types=[pltpu.VMEM_SHARED((max_len * d,), jnp.float32)],
    )
    def body(w_hbm, out_hbm, w_spmem):
        c = lax.axis_index("c")
        s = lax.axis_index("s")

        @pl.when(s == 0)
        def _stage():
            pltpu.sync_copy(w_hbm, w_spmem)

        plsc.subcore_barrier()

        wid = c * _NUM_SUBCORES + s
        for t in range(per_worker):
            i = wid * per_worker + t
            src_start = pl.multiple_of((offset - i) * d, d)
            dst_start = pl.multiple_of(i * slice_words, slice_words)
            pltpu.sync_copy(
                w_spmem.at[pl.ds(src_start, slice_words)],
                out_hbm.at[pl.ds(dst_start, slice_words)],
            )

    out_flat = body(weight.reshape(-1))
    return out_flat.reshape(q_len, k_len, d)


# speedup vs baseline: 1.6271x; 1.0057x over previous
import functools

import jax
import jax.numpy as jnp
from jax import lax
from jax.experimental import pallas as pl
from jax.experimental.pallas import tpu as pltpu
from jax.experimental.pallas import tpu_sc as plsc

_NUM_CORES = 2
_NUM_SUBCORES = 16


def kernel(q, k, weight):
    q_len = q.shape[0]
    k_len = k.shape[0]
    max_len, d = weight.shape
    offset = max_len // 2 + max_len % 2

    n_workers = _NUM_CORES * _NUM_SUBCORES
    per_worker = q_len // n_workers
    slice_words = k_len * d

    mesh = plsc.VectorSubcoreMesh(core_axis_name="c", subcore_axis_name="s")

    @functools.partial(
        pl.kernel,
        out_type=jax.ShapeDtypeStruct((q_len * k_len * d,), jnp.float32),
        mesh=mesh,
        scratch_types=[
            pltpu.VMEM_SHARED((max_len * d,), jnp.float32),
            pltpu.SemaphoreType.DMA,
        ],
    )
    def body(w_hbm, out_hbm, w_spmem, sem):
        c = lax.axis_index("c")
        s = lax.axis_index("s")

        @pl.when(s == 0)
        def _stage():
            pltpu.sync_copy(w_hbm, w_spmem)

        plsc.subcore_barrier()

        wid = c * _NUM_SUBCORES + s
        copies = []
        for t in range(per_worker):
            i = wid * per_worker + t
            src_start = pl.multiple_of((offset - i) * d, d)
            dst_start = pl.multiple_of(i * slice_words, slice_words)
            copies.append(
                pltpu.async_copy(
                    w_spmem.at[pl.ds(src_start, slice_words)],
                    out_hbm.at[pl.ds(dst_start, slice_words)],
                    sem,
                )
            )
        for cp in copies:
            cp.wait()

    out_flat = body(weight.reshape(-1))
    return out_flat.reshape(q_len, k_len, d)
